# baseline (speedup 1.0000x reference)
import math

import jax
import jax.numpy as jnp
from jax import lax
from jax.experimental import pallas as pl
from jax.experimental.pallas import tpu as pltpu
from jax.experimental.pallas import tpu_sc as plsc

BETA_ = 0.99
LOG_BETA_ = math.log(BETA_)
L = 16
GW = 128


def _make_sc_loss(N, C):
    NS = 16
    NC = 2
    HPT = N // NS
    HK = HPT // L
    NG = C // GW
    NQ = (NC * NS) // NG
    QS = N // NQ
    K2 = QS // L
    CB = C // NS

    mesh = plsc.VectorSubcoreMesh(core_axis_name="c", subcore_axis_name="s",
                                  num_cores=NC, num_subcores=NS)

    def body(preds_hbm, y_hbm, m_hbm, out_hbm,
             y_v, m_v, yq_v, mq_v, list_v, ylist_v, mlist_v, seg0, seg1,
             hist1d, histr_v, cslice_v, red_v, cnt1d_v, w_v, pp_v, pall_v,
             out_v, sh, dsem, csem):
        cid = lax.axis_index("c")
        sid = lax.axis_index("s")
        wid = cid * NS + sid
        lane = lax.iota(jnp.int32, L)
        hbase = sid * HPT
        q = wid // NG
        g = wid % NG
        gcol = pl.multiple_of(g * GW, GW)
        preds2d = preds_hbm.reshape(N, C)

        pltpu.sync_copy(y_hbm.at[pl.ds(hbase, HPT)], y_v)
        pltpu.sync_copy(m_hbm.at[pl.ds(hbase, HPT)], m_v)
        pltpu.sync_copy(y_hbm.at[pl.ds(q * QS, QS)], yq_v)
        pltpu.sync_copy(m_hbm.at[pl.ds(q * QS, QS)], mq_v)

        zi = jnp.zeros((L,), jnp.int32)
        zf = jnp.zeros((L,), jnp.float32)

        @pl.loop(0, K2, unroll=8)
        def _(i):
            list_v[pl.ds(i * L, L)] = zi
            ylist_v[pl.ds(i * L, L)] = zi
            mlist_v[pl.ds(i * L, L)] = zf

        def comp_body(i, base_v):
            yv = yq_v[pl.ds(i * L, L)]
            mv = mq_v[pl.ds(i * L, L)]
            msk = lax.shift_right_logical(yv, 7) == g
            cs = plsc.cumsum(msk.astype(jnp.int32))
            slots = base_v + cs - 1
            plsc.store_scatter(list_v, [slots], q * QS + i * L + lane,
                               mask=msk)
            plsc.store_scatter(ylist_v, [slots], yv, mask=msk)
            plsc.store_scatter(mlist_v, [slots], mv, mask=msk)
            return base_v + plsc.all_reduce_population_count(msk)

        base_v = lax.fori_loop(0, K2, comp_body, jnp.zeros((L,), jnp.int32))
        n = base_v[0]
        nch = lax.shift_right_logical(n + (GW - 1), 7)
        npairs = lax.shift_right_logical(nch + 1, 1)
        nch_e = npairs * 2

        def fire(k, buf):
            pltpu.async_copy(
                preds2d.at[list_v.at[pl.ds(pl.multiple_of(k * GW, GW), GW)],
                           pl.ds(gcol, GW)], buf, dsem)

        def drain(buf):
            pltpu.make_async_copy(
                preds2d.at[list_v.at[pl.ds(0, GW)], pl.ds(gcol, GW)],
                buf, dsem).wait()

        @pl.when(npairs > 0)
        def _():
            fire(0, seg0)
            fire(1, seg1)

        @pl.loop(0, NS * C // L, unroll=8)
        def _(i):
            hist1d[pl.ds(i * L, L)] = zf

        @pl.loop(0, HK, unroll=4)
        def _(c):
            yv = y_v[pl.ds(c * L, L)]
            mv = m_v[pl.ds(c * L, L)]
            plsc.addupdate_scatter(hist1d, [lane * C + yv], mv)

        @pl.loop(0, C // L)
        def _(c16):
            acc = zf
            for l in range(L):
                acc = acc + hist1d[pl.ds(l * C + c16 * L, L)]
            histr_v[pl.ds(c16 * L, L)] = acc

        pltpu.sync_copy(histr_v, sh.at[sid])
        plsc.subcore_barrier()

        for s in range(NS):
            pltpu.async_copy(sh.at[s, pl.ds(sid * CB, CB)], cslice_v.at[s],
                             csem)
        for s in range(NS):
            pltpu.make_async_copy(sh.at[s, pl.ds(sid * CB, CB)],
                                  cslice_v.at[s], csem).wait()
        for k in range(CB // L):
            acc = zf
            for s in range(NS):
                acc = acc + cslice_v[s, pl.ds(k * L, L)]
            red_v[pl.ds(k * L, L)] = acc
        pltpu.sync_copy(red_v, sh.at[NS + sid, pl.ds(0, CB)])
        plsc.subcore_barrier()

        for s in range(NS):
            pltpu.async_copy(sh.at[NS + s, pl.ds(0, CB)],
                             cnt1d_v.at[pl.ds(s * CB, CB)], csem)
        for s in range(NS):
            pltpu.make_async_copy(sh.at[NS + s, pl.ds(0, CB)],
                                  cnt1d_v.at[pl.ds(s * CB, CB)], csem).wait()

        @pl.loop(0, C // L, unroll=4)
        def _(c16):
            cnt = cnt1d_v[pl.ds(c16 * L, L)]
            bpow = jnp.exp(cnt * jnp.float32(LOG_BETA_))
            w = jnp.float32(1.0 - BETA_) / ((1.0 - bpow) + jnp.float32(1e-8))
            w_v[pl.ds(c16 * L, L)] = w

        def consume(k, buf, a1, a2):
            drain(buf)
            for v in range(GW // L):
                ys = ylist_v[pl.ds(k * GW + v * L, L)]
                ms = mlist_v[pl.ds(k * GW + v * L, L)]
                wv = plsc.load_gather(w_v, [ys]) * ms
                pv = plsc.load_gather(buf, [v * L + lane, ys & (GW - 1)])
                a1 = a1 + pv * wv
                a2 = a2 + wv
            return a1, a2

        def pair(p, carry):
            a1, a2 = carry
            a1, a2 = consume(2 * p, seg0, a1, a2)

            @pl.when(2 * p + 2 < nch_e)
            def _():
                fire(2 * p + 2, seg0)

            a1, a2 = consume(2 * p + 1, seg1, a1, a2)

            @pl.when(2 * p + 3 < nch_e)
            def _():
                fire(2 * p + 3, seg1)

            return a1, a2

        z = jnp.zeros((L,), jnp.float32)
        a1, a2 = lax.fori_loop(0, npairs, pair, (z, z))
        pp_v[pl.ds(0, L)] = a1
        pp_v[pl.ds(L, L)] = a2
        pltpu.sync_copy(pp_v, sh.at[sid, pl.ds(0, 2 * L)])
        plsc.subcore_barrier()

        @pl.when(sid == 0)
        def _():
            for s in range(NS):
                pltpu.async_copy(sh.at[s, pl.ds(0, 2 * L)],
                                 pall_v.at[pl.ds(s * 2 * L, 2 * L)], csem)
            for s in range(NS):
                pltpu.make_async_copy(sh.at[s, pl.ds(0, 2 * L)],
                                      pall_v.at[pl.ds(s * 2 * L, 2 * L)],
                                      csem).wait()

            def rbody(t, carry):
                b1, b2 = carry
                return (b1 + pall_v[pl.ds(t * 2 * L, L)],
                        b2 + pall_v[pl.ds(t * 2 * L + L, L)])

            b1, b2 = lax.fori_loop(0, NS, rbody, (z, z))
            v1 = jnp.full((L,), jnp.sum(b1), dtype=jnp.float32)
            v2 = jnp.full((L,), jnp.sum(b2), dtype=jnp.float32)
            out_v[pl.ds(0, L)] = v1
            out_v[pl.ds(L, L)] = v2
            pltpu.sync_copy(out_v, out_hbm.at[cid])

    return pl.kernel(
        body,
        out_type=jax.ShapeDtypeStruct((NC, 2 * L), jnp.float32),
        mesh=mesh,
        compiler_params=pltpu.CompilerParams(needs_layout_passes=False),
        scratch_types=[
            pltpu.VMEM((HPT,), jnp.int32),
            pltpu.VMEM((HPT,), jnp.float32),
            pltpu.VMEM((QS,), jnp.int32),
            pltpu.VMEM((QS,), jnp.float32),
            pltpu.VMEM((QS,), jnp.int32),
            pltpu.VMEM((QS,), jnp.int32),
            pltpu.VMEM((QS,), jnp.float32),
            pltpu.VMEM((GW, GW), jnp.float32),
            pltpu.VMEM((GW, GW), jnp.float32),
            pltpu.VMEM((NS * C,), jnp.float32),
            pltpu.VMEM((C,), jnp.float32),
            pltpu.VMEM((NS, CB), jnp.float32),
            pltpu.VMEM((CB,), jnp.float32),
            pltpu.VMEM((C,), jnp.float32),
            pltpu.VMEM((C,), jnp.float32),
            pltpu.VMEM((2 * L,), jnp.float32),
            pltpu.VMEM((2 * L * NS,), jnp.float32),
            pltpu.VMEM((2 * L,), jnp.float32),
            pltpu.VMEM_SHARED((2 * NS, C), jnp.float32),
            pltpu.SemaphoreType.DMA,
            pltpu.SemaphoreType.DMA,
        ],
    )


def kernel(preds, labels, pad_mask):
    B, S, C = preds.shape
    N = B * S
    y = labels.reshape(-1).astype(jnp.int32)
    m = pad_mask.reshape(-1).astype(jnp.float32)
    out = _make_sc_loss(N, C)(preds, y, m)
    s1 = out[0, 0] + out[1, 0]
    s2 = out[0, L] + out[1, L]
    return -s1 / s2


# speedup vs baseline: 1.3764x; 1.3764x over previous
import math

import jax
import jax.numpy as jnp
from jax import lax
from jax.experimental import pallas as pl
from jax.experimental.pallas import tpu as pltpu
from jax.experimental.pallas import tpu_sc as plsc

BETA_ = 0.99
LOG_BETA_ = math.log(BETA_)
L = 16
CH = 32


def _make_sc_loss(N, C, S):
    NS = 16
    NC = 2
    HPT = N // NS
    HK = HPT // L
    SPT = N // (NS * NC)
    NCH = SPT // CH
    CB = C // NS

    mesh = plsc.VectorSubcoreMesh(core_axis_name="c", subcore_axis_name="s",
                                  num_cores=NC, num_subcores=NS)

    def body(preds_hbm, y_hbm, m_hbm, out_hbm,
             y_v, m_v, y2_v, m2_v, rows0, rows1, hist1d, histr_v, cslice_v,
             red_v, cnt1d_v, w_v, pp_v, pall_v, out_v, sh, dsem, csem):
        cid = lax.axis_index("c")
        sid = lax.axis_index("s")
        lane = lax.iota(jnp.int32, L)
        hbase = sid * HPT
        gbase = (cid * NS + sid) * SPT
        bt = gbase // S
        s0 = gbase % S

        def fire(k, buf):
            pltpu.async_copy(
                preds_hbm.at[bt, pl.ds(pl.multiple_of(s0 + k * CH, CH), CH)],
                buf, dsem)

        def drain(buf):
            pltpu.make_async_copy(
                preds_hbm.at[bt, pl.ds(pl.multiple_of(s0, CH), CH)],
                buf, dsem).wait()

        fire(0, rows0)
        fire(1, rows1)

        pltpu.sync_copy(y_hbm.at[pl.ds(hbase, HPT)], y_v)
        pltpu.sync_copy(m_hbm.at[pl.ds(hbase, HPT)], m_v)
        pltpu.sync_copy(y_hbm.at[pl.ds(gbase, SPT)], y2_v)
        pltpu.sync_copy(m_hbm.at[pl.ds(gbase, SPT)], m2_v)

        @pl.loop(0, NS * C // L, unroll=8)
        def _(i):
            hist1d[pl.ds(i * L, L)] = jnp.zeros((L,), jnp.float32)

        @pl.loop(0, HK, unroll=4)
        def _(c):
            yv = y_v[pl.ds(c * L, L)]
            mv = m_v[pl.ds(c * L, L)]
            plsc.addupdate_scatter(hist1d, [lane * C + yv], mv)

        @pl.loop(0, C // L)
        def _(c16):
            acc = jnp.zeros((L,), jnp.float32)
            for l in range(L):
                acc = acc + hist1d[pl.ds(l * C + c16 * L, L)]
            histr_v[pl.ds(c16 * L, L)] = acc

        pltpu.sync_copy(histr_v, sh.at[sid])
        plsc.subcore_barrier()

        for s in range(NS):
            pltpu.async_copy(sh.at[s, pl.ds(sid * CB, CB)], cslice_v.at[s],
                             csem)
        for s in range(NS):
            pltpu.make_async_copy(sh.at[s, pl.ds(sid * CB, CB)],
                                  cslice_v.at[s], csem).wait()
        for k in range(CB // L):
            acc = jnp.zeros((L,), jnp.float32)
            for s in range(NS):
                acc = acc + cslice_v[s, pl.ds(k * L, L)]
            red_v[pl.ds(k * L, L)] = acc
        pltpu.sync_copy(red_v, sh.at[NS + sid, pl.ds(0, CB)])
        plsc.subcore_barrier()

        for s in range(NS):
            pltpu.async_copy(sh.at[NS + s, pl.ds(0, CB)],
                             cnt1d_v.at[pl.ds(s * CB, CB)], csem)
        for s in range(NS):
            pltpu.make_async_copy(sh.at[NS + s, pl.ds(0, CB)],
                                  cnt1d_v.at[pl.ds(s * CB, CB)], csem).wait()

        @pl.loop(0, C // L, unroll=4)
        def _(c16):
            cnt = cnt1d_v[pl.ds(c16 * L, L)]
            bpow = jnp.exp(cnt * jnp.float32(LOG_BETA_))
            w = jnp.float32(1.0 - BETA_) / ((1.0 - bpow) + jnp.float32(1e-8))
            w_v[pl.ds(c16 * L, L)] = w

        def consume(k, buf, a1, a2):
            drain(buf)
            for v in range(CH // L):
                yv = y2_v[pl.ds(k * CH + v * L, L)]
                mv = m2_v[pl.ds(k * CH + v * L, L)]
                wv = plsc.load_gather(w_v, [yv]) * mv
                pv = plsc.load_gather(buf, [v * L + lane, yv])
                a1 = a1 + pv * wv
                a2 = a2 + wv
            return a1, a2

        def pair(p, carry):
            a1, a2 = carry
            a1, a2 = consume(2 * p, rows0, a1, a2)

            @pl.when(2 * p + 2 < NCH)
            def _():
                fire(2 * p + 2, rows0)

            a1, a2 = consume(2 * p + 1, rows1, a1, a2)

            @pl.when(2 * p + 3 < NCH)
            def _():
                fire(2 * p + 3, rows1)

            return a1, a2

        z = jnp.zeros((L,), jnp.float32)
        a1, a2 = lax.fori_loop(0, NCH // 2, pair, (z, z))
        pp_v[pl.ds(0, L)] = a1
        pp_v[pl.ds(L, L)] = a2
        pltpu.sync_copy(pp_v, sh.at[sid, pl.ds(0, 2 * L)])
        plsc.subcore_barrier()

        @pl.when(sid == 0)
        def _():
            for s in range(NS):
                pltpu.async_copy(sh.at[s, pl.ds(0, 2 * L)],
                                 pall_v.at[pl.ds(s * 2 * L, 2 * L)], csem)
            for s in range(NS):
                pltpu.make_async_copy(sh.at[s, pl.ds(0, 2 * L)],
                                      pall_v.at[pl.ds(s * 2 * L, 2 * L)],
                                      csem).wait()

            def rbody(t, carry):
                b1, b2 = carry
                return (b1 + pall_v[pl.ds(t * 2 * L, L)],
                        b2 + pall_v[pl.ds(t * 2 * L + L, L)])

            b1, b2 = lax.fori_loop(0, NS, rbody, (z, z))
            v1 = jnp.full((L,), jnp.sum(b1), dtype=jnp.float32)
            v2 = jnp.full((L,), jnp.sum(b2), dtype=jnp.float32)
            out_v[pl.ds(0, L)] = v1
            out_v[pl.ds(L, L)] = v2
            pltpu.sync_copy(out_v, out_hbm.at[cid])

    return pl.kernel(
        body,
        out_type=jax.ShapeDtypeStruct((NC, 2 * L), jnp.float32),
        mesh=mesh,
        compiler_params=pltpu.CompilerParams(needs_layout_passes=False),
        scratch_types=[
            pltpu.VMEM((HPT,), jnp.int32),
            pltpu.VMEM((HPT,), jnp.float32),
            pltpu.VMEM((SPT,), jnp.int32),
            pltpu.VMEM((SPT,), jnp.float32),
            pltpu.VMEM((CH, C), jnp.float32),
            pltpu.VMEM((CH, C), jnp.float32),
            pltpu.VMEM((NS * C,), jnp.float32),
            pltpu.VMEM((C,), jnp.float32),
            pltpu.VMEM((NS, CB), jnp.float32),
            pltpu.VMEM((CB,), jnp.float32),
            pltpu.VMEM((C,), jnp.float32),
            pltpu.VMEM((C,), jnp.float32),
            pltpu.VMEM((2 * L,), jnp.float32),
            pltpu.VMEM((2 * L * NS,), jnp.float32),
            pltpu.VMEM((2 * L,), jnp.float32),
            pltpu.VMEM_SHARED((2 * NS, C), jnp.float32),
            pltpu.SemaphoreType.DMA,
            pltpu.SemaphoreType.DMA,
        ],
    )


def kernel(preds, labels, pad_mask):
    B, S, C = preds.shape
    N = B * S
    y = labels.reshape(-1).astype(jnp.int32)
    m = pad_mask.reshape(-1).astype(jnp.float32)
    out = _make_sc_loss(N, C, S)(preds, y, m)
    s1 = out[0, 0] + out[1, 0]
    s2 = out[0, L] + out[1, L]
    return -s1 / s2
